# baseline (speedup 1.0000x reference)
import functools

import jax
import jax.numpy as jnp
from jax import lax
from jax.experimental import pallas as pl
from jax.experimental.pallas import tpu as pltpu
from jax.experimental.pallas import tpu_sc as plsc

VOCAB = 1000
EMBED_DIM = 128
NTOK = 4096 * 20
NW = 32
ROWS_PER_W = NTOK // NW
CHUNK = 64
NCHUNK = ROWS_PER_W // CHUNK


def _proj_table_kernel(emb_ref, w_ref, b_ref, p_ref):
    p_ref[...] = lax.dot_general(
        emb_ref[...], w_ref[...],
        (((1,), (1,)), ((), ())),
        preferred_element_type=jnp.float32,
    ) + b_ref[...]


_mesh = plsc.VectorSubcoreMesh(
    core_axis_name="c", subcore_axis_name="s", num_cores=2, num_subcores=16
)


@functools.partial(
    pl.kernel,
    out_type=jax.ShapeDtypeStruct((NTOK, VOCAB), jnp.float32),
    mesh=_mesh,
    scratch_types=[
        pltpu.VMEM((CHUNK,), jnp.int32),
        pltpu.VMEM((CHUNK, VOCAB), jnp.float32),
        pltpu.SemaphoreType.DMA,
    ],
    compiler_params=pltpu.CompilerParams(use_tc_tiling_on_sc=False),
)
def _gather_rows(table_hbm, idx_hbm, out_hbm, idx_v, rows_v, sem):
    wid = lax.axis_index("s") * 2 + lax.axis_index("c")
    w_base = wid * ROWS_PER_W

    def body(i, carry):
        base = w_base + i * CHUNK
        pltpu.sync_copy(idx_hbm.at[pl.ds(base, CHUNK)], idx_v)
        pltpu.async_copy(table_hbm.at[idx_v], rows_v, sem).wait()
        pltpu.sync_copy(rows_v, out_hbm.at[pl.ds(base, CHUNK)])
        return carry

    lax.fori_loop(0, NCHUNK, body, 0)


def kernel(x, emb_table, W, b):
    P = pl.pallas_call(
        _proj_table_kernel,
        out_shape=jax.ShapeDtypeStruct((VOCAB, VOCAB), jnp.float32),
    )(emb_table, W, b.reshape(1, VOCAB))
    idx = x.reshape(-1).astype(jnp.int32)
    logits = _gather_rows(P, idx)
    return logits.reshape(x.shape[0], x.shape[1], VOCAB)


# speedup vs baseline: 1.0368x; 1.0368x over previous
import functools

import jax
import jax.numpy as jnp
from jax import lax
from jax.experimental import pallas as pl
from jax.experimental.pallas import tpu as pltpu
from jax.experimental.pallas import tpu_sc as plsc

VOCAB = 1000
EMBED_DIM = 128
NTOK = 4096 * 20
NW = 32
ROWS_PER_W = NTOK // NW
CHUNK = 64
NCHUNK = ROWS_PER_W // CHUNK


def _proj_table_kernel(emb_ref, w_ref, b_ref, p_ref):
    p_ref[...] = lax.dot_general(
        emb_ref[...], w_ref[...],
        (((1,), (1,)), ((), ())),
        preferred_element_type=jnp.float32,
    ) + b_ref[...]


_mesh = plsc.VectorSubcoreMesh(
    core_axis_name="c", subcore_axis_name="s", num_cores=2, num_subcores=16
)


@functools.partial(
    pl.kernel,
    out_type=jax.ShapeDtypeStruct((NTOK, VOCAB), jnp.float32),
    mesh=_mesh,
    scratch_types=[
        pltpu.VMEM((ROWS_PER_W,), jnp.int32),
        pltpu.VMEM((CHUNK, VOCAB), jnp.float32),
        pltpu.VMEM((CHUNK, VOCAB), jnp.float32),
        pltpu.SemaphoreType.DMA,
        pltpu.SemaphoreType.DMA,
        pltpu.SemaphoreType.DMA,
        pltpu.SemaphoreType.DMA,
    ],
    compiler_params=pltpu.CompilerParams(use_tc_tiling_on_sc=False),
)
def _gather_rows(table_hbm, idx_hbm, out_hbm, idx_v, rows0, rows1,
                 sg0, sg1, sw0, sw1):
    wid = lax.axis_index("s") * 2 + lax.axis_index("c")
    w_base = wid * ROWS_PER_W
    rows = (rows0, rows1)
    sg = (sg0, sg1)
    sw = (sw0, sw1)

    pltpu.sync_copy(idx_hbm.at[pl.ds(w_base, ROWS_PER_W)], idx_v)

    def start_gather(i, b):
        pltpu.async_copy(table_hbm.at[idx_v.at[pl.ds(i * CHUNK, CHUNK)]],
                         rows[b], sg[b])

    def wait_gather(i, b):
        pltpu.make_async_copy(table_hbm.at[idx_v.at[pl.ds(i * CHUNK, CHUNK)]],
                              rows[b], sg[b]).wait()

    def start_write(i, b):
        pltpu.async_copy(rows[b], out_hbm.at[pl.ds(w_base + i * CHUNK, CHUNK)],
                         sw[b])

    def wait_write(i, b):
        pltpu.make_async_copy(rows[b],
                              out_hbm.at[pl.ds(w_base + i * CHUNK, CHUNK)],
                              sw[b]).wait()

    start_gather(0, 0)
    start_gather(1, 1)
    wait_gather(0, 0)
    start_write(0, 0)
    wait_write(0, 0)
    start_gather(2, 0)
    wait_gather(1, 1)
    start_write(1, 1)

    def body(j, carry):
        i0 = 2 * j
        wait_write(i0 - 1, 1)
        start_gather(i0 + 1, 1)
        wait_gather(i0, 0)
        start_write(i0, 0)
        i1 = i0 + 1
        wait_write(i1 - 1, 0)
        start_gather(i1 + 1, 0)
        wait_gather(i1, 1)
        start_write(i1, 1)
        return carry

    lax.fori_loop(1, NCHUNK // 2 - 1, body, 0)

    i0 = NCHUNK - 2
    wait_write(i0 - 1, 1)
    start_gather(i0 + 1, 1)
    wait_gather(i0, 0)
    start_write(i0, 0)
    wait_gather(i0 + 1, 1)
    start_write(i0 + 1, 1)
    wait_write(i0, 0)
    wait_write(i0 + 1, 1)


def kernel(x, emb_table, W, b):
    P = pl.pallas_call(
        _proj_table_kernel,
        out_shape=jax.ShapeDtypeStruct((VOCAB, VOCAB), jnp.float32),
    )(emb_table, W, b.reshape(1, VOCAB))
    idx = x.reshape(-1).astype(jnp.int32)
    logits = _gather_rows(P, idx)
    return logits.reshape(x.shape[0], x.shape[1], VOCAB)
